# baseline (speedup 1.0000x reference)
import functools

import jax
import jax.numpy as jnp
from jax.experimental import pallas as pl
from jax.experimental.pallas import tpu as pltpu

N_POINTS = 16384
N_SAMPLE = 8192
RADIUS = 0.2
K = 32
HIDDEN = 64
OUT = 128

QBLK = 256


FPS_R = 128
FPS_C = 128


def _fps_kernel(xs_ref, ys_ref, zs_ref, xsm_ref, ysm_ref, zsm_ref, sel_ref):
    rows = jax.lax.broadcasted_iota(jnp.int32, (FPS_R, FPS_C), 0)
    cols = jax.lax.broadcasted_iota(jnp.int32, (FPS_R, FPS_C), 1)
    lin = rows * FPS_C + cols
    xs = xs_ref[...]
    ys = ys_ref[...]
    zs = zs_ref[...]
    sel_ref[0] = 0

    def body(i, state):
        last, dists = state
        px = xsm_ref[last]
        py = ysm_ref[last]
        pz = zsm_ref[last]
        dx = xs - px
        dy = ys - py
        dz = zs - pz
        d = (dx * dx + dz * dz) + dy * dy
        dists = jnp.minimum(dists, d)
        m = jnp.max(dists)
        nxt = jnp.min(jnp.where(dists == m, lin, jnp.int32(N_POINTS)))
        sel_ref[i] = nxt
        return (nxt, dists)

    dists0 = jnp.full((FPS_R, FPS_C), jnp.inf, dtype=jnp.float32)
    jax.lax.fori_loop(1, N_SAMPLE, body, (jnp.int32(0), dists0))


def _fps(pos, n_sample):
    xs = pos[:, 0].reshape(FPS_R, FPS_C)
    ys = pos[:, 1].reshape(FPS_R, FPS_C)
    zs = pos[:, 2].reshape(FPS_R, FPS_C)
    return pl.pallas_call(
        _fps_kernel,
        in_specs=[
            pl.BlockSpec(memory_space=pltpu.VMEM),
            pl.BlockSpec(memory_space=pltpu.VMEM),
            pl.BlockSpec(memory_space=pltpu.VMEM),
            pl.BlockSpec(memory_space=pltpu.SMEM),
            pl.BlockSpec(memory_space=pltpu.SMEM),
            pl.BlockSpec(memory_space=pltpu.SMEM),
        ],
        out_specs=pl.BlockSpec(memory_space=pltpu.SMEM),
        out_shape=jax.ShapeDtypeStruct((n_sample,), jnp.int32),
    )(xs, ys, zs, pos[:, 0], pos[:, 1], pos[:, 2])


QB = 128
DEPTH = 5
NCH = 128
CL = 128
R2 = RADIUS * RADIUS
BIGI = N_POINTS
INF = float("inf")


def _select_kernel(qx_ref, qy_ref, qz_ref, xs_ref, ys_ref, zs_ref,
                   nbr_ref, val_ref):
    qx = qx_ref[...].reshape(QB, 1, 1)
    qy = qy_ref[...].reshape(QB, 1, 1)
    qz = qz_ref[...].reshape(QB, 1, 1)
    xs = xs_ref[...].reshape(1, NCH, CL)
    ys = ys_ref[...].reshape(1, NCH, CL)
    zs = zs_ref[...].reshape(1, NCH, CL)

    lane3 = jax.lax.broadcasted_iota(jnp.int32, (QB, NCH, CL), 2)
    chunk2 = jax.lax.broadcasted_iota(jnp.int32, (QB, NCH), 1)
    lane32 = jax.lax.broadcasted_iota(jnp.int32, (QB, K), 1)

    def d2_full():
        dx = qx - xs
        dy = qy - ys
        dz = qz - zs
        return (dx * dx + dy * dy) + dz * dz

    D = d2_full()
    vals_l, idxs_l = [], []
    for _ in range(DEPTH):
        m = jnp.min(D, axis=2)
        eq = D == m[:, :, None]
        jl = jnp.min(jnp.where(eq, lane3, BIGI), axis=2)
        D = jnp.where(lane3 == jl[:, :, None], INF, D)
        vals_l.append(m)
        idxs_l.append(chunk2 * CL + jl)
    vals = jnp.stack(vals_l, axis=1)
    idxs = jnp.stack(idxs_l, axis=1)
    vlast = vals_l[-1]

    def p2_body(t, state):
        v, nbr, val, _ = state
        m2 = jnp.min(v, axis=(1, 2))
        eq = v == m2[:, None, None]
        j = jnp.min(jnp.where(eq, idxs, BIGI), axis=(1, 2))
        v = jnp.where(eq & (idxs == j[:, None, None]), INF, v)
        nbr = jnp.where(lane32 == t, j[:, None], nbr)
        ok = (m2 <= R2).astype(jnp.int32)
        val = jnp.where(lane32 == t, ok[:, None], val)
        return (v, nbr, val, m2)

    nbr0 = jnp.zeros((QB, K), jnp.int32)
    val0 = jnp.zeros((QB, K), jnp.int32)
    _, nbr, val, t32 = jax.lax.fori_loop(
        0, K, p2_body, (vals, nbr0, val0, jnp.zeros((QB,), jnp.float32)))
    nbr_ref[...] = nbr
    val_ref[...] = val

    unsafe = jnp.any(vlast <= t32[:, None])

    @pl.when(unsafe)
    def _fallback():
        lin = chunk2[:, :, None] * CL + lane3
        Df = d2_full()

        def fb_body(t, state):
            Dc, nbr, val = state
            m2 = jnp.min(Dc, axis=(1, 2))
            eq = Dc == m2[:, None, None]
            j = jnp.min(jnp.where(eq, lin, BIGI), axis=(1, 2))
            Dc = jnp.where(lin == j[:, None, None], INF, Dc)
            nbr = jnp.where(lane32 == t, j[:, None], nbr)
            ok = (m2 <= R2).astype(jnp.int32)
            val = jnp.where(lane32 == t, ok[:, None], val)
            return (Dc, nbr, val)

        _, nbr_f, val_f = jax.lax.fori_loop(0, K, fb_body, (Df, nbr0, val0))
        nbr_ref[...] = nbr_f
        val_ref[...] = val_f


def _radius_query(pos, q, r, k):
    s = q.shape[0]
    qx = q[:, 0].reshape(s, 1)
    qy = q[:, 1].reshape(s, 1)
    qz = q[:, 2].reshape(s, 1)
    xs = pos[:, 0].reshape(NCH, CL)
    ys = pos[:, 1].reshape(NCH, CL)
    zs = pos[:, 2].reshape(NCH, CL)
    grid = s // QB
    nbr, val = pl.pallas_call(
        _select_kernel,
        grid=(grid,),
        in_specs=[
            pl.BlockSpec((QB, 1), lambda i: (i, 0)),
            pl.BlockSpec((QB, 1), lambda i: (i, 0)),
            pl.BlockSpec((QB, 1), lambda i: (i, 0)),
            pl.BlockSpec((NCH, CL), lambda i: (0, 0)),
            pl.BlockSpec((NCH, CL), lambda i: (0, 0)),
            pl.BlockSpec((NCH, CL), lambda i: (0, 0)),
        ],
        out_specs=[
            pl.BlockSpec((QB, K), lambda i: (i, 0)),
            pl.BlockSpec((QB, K), lambda i: (i, 0)),
        ],
        out_shape=[
            jax.ShapeDtypeStruct((s, k), jnp.int32),
            jax.ShapeDtypeStruct((s, k), jnp.int32),
        ],
    )(qx, qy, qz, xs, ys, zs)
    return nbr, val != 0


def _edge_kernel(relj_ref, reli_ref, mask_ref, w1_ref, b1_ref, w2_ref, b2_ref,
                 out_ref):
    rel = relj_ref[...] - reli_ref[...]
    h = jnp.maximum(rel @ w1_ref[...] + b1_ref[...], 0.0)
    msg = h @ w2_ref[...] + b2_ref[...]
    neg = jnp.float32(-jnp.inf)
    msg = jnp.where(mask_ref[...], msg, neg)
    msg = msg.reshape(QBLK, K, OUT)
    red = jnp.max(msg, axis=1)
    out_ref[...] = jnp.where(jnp.isfinite(red), red, 0.0)


def kernel(pos, batch, W1, b1, W2, b2):
    idx = _fps(pos, N_SAMPLE)
    q = pos[idx]
    nbr, valid = _radius_query(pos, q, RADIUS, K)

    col = nbr.reshape(-1)
    posj = pos[col]
    posi = jnp.repeat(pos[:N_SAMPLE], K, axis=0)
    mask = valid.reshape(-1, 1)

    grid = N_SAMPLE // QBLK
    out_top = pl.pallas_call(
        _edge_kernel,
        grid=(grid,),
        in_specs=[
            pl.BlockSpec((QBLK * K, 3), lambda i: (i, 0)),
            pl.BlockSpec((QBLK * K, 3), lambda i: (i, 0)),
            pl.BlockSpec((QBLK * K, 1), lambda i: (i, 0)),
            pl.BlockSpec((3, HIDDEN), lambda i: (0, 0)),
            pl.BlockSpec((HIDDEN,), lambda i: (0,)),
            pl.BlockSpec((HIDDEN, OUT), lambda i: (0, 0)),
            pl.BlockSpec((OUT,), lambda i: (0,)),
        ],
        out_specs=pl.BlockSpec((QBLK, OUT), lambda i: (i, 0)),
        out_shape=jax.ShapeDtypeStruct((N_SAMPLE, OUT), jnp.float32),
    )(posj, posi, mask, W1, b1, W2, b2)

    return jnp.concatenate(
        [out_top, jnp.zeros((N_POINTS - N_SAMPLE, OUT), jnp.float32)], axis=0)


# speedup vs baseline: 1.1735x; 1.1735x over previous
import functools

import jax
import jax.numpy as jnp
from jax import lax
from jax.experimental import pallas as pl
from jax.experimental.pallas import tpu as pltpu
from jax.experimental.pallas import tpu_sc as plsc

N_POINTS = 16384
N_SAMPLE = 8192
RADIUS = 0.2
K = 32
HIDDEN = 64
OUT = 128

QBLK = 256


FPS_R = 128
FPS_C = 128


def _fps_kernel(xs_ref, ys_ref, zs_ref, xsm_ref, ysm_ref, zsm_ref, sel_ref):
    rows = jax.lax.broadcasted_iota(jnp.int32, (FPS_R, FPS_C), 0)
    cols = jax.lax.broadcasted_iota(jnp.int32, (FPS_R, FPS_C), 1)
    lin = rows * FPS_C + cols
    xs = xs_ref[...]
    ys = ys_ref[...]
    zs = zs_ref[...]
    sel_ref[0] = 0

    def body(i, state):
        last, dists = state
        px = xsm_ref[last]
        py = ysm_ref[last]
        pz = zsm_ref[last]
        dx = xs - px
        dy = ys - py
        dz = zs - pz
        d = (dx * dx + dz * dz) + dy * dy
        dists = jnp.minimum(dists, d)
        m = jnp.max(dists)
        nxt = jnp.min(jnp.where(dists == m, lin, jnp.int32(N_POINTS)))
        sel_ref[i] = nxt
        return (nxt, dists)

    dists0 = jnp.full((FPS_R, FPS_C), jnp.inf, dtype=jnp.float32)
    jax.lax.fori_loop(1, N_SAMPLE, body, (jnp.int32(0), dists0))


def _fps(pos, n_sample):
    xs = pos[:, 0].reshape(FPS_R, FPS_C)
    ys = pos[:, 1].reshape(FPS_R, FPS_C)
    zs = pos[:, 2].reshape(FPS_R, FPS_C)
    return pl.pallas_call(
        _fps_kernel,
        in_specs=[
            pl.BlockSpec(memory_space=pltpu.VMEM),
            pl.BlockSpec(memory_space=pltpu.VMEM),
            pl.BlockSpec(memory_space=pltpu.VMEM),
            pl.BlockSpec(memory_space=pltpu.SMEM),
            pl.BlockSpec(memory_space=pltpu.SMEM),
            pl.BlockSpec(memory_space=pltpu.SMEM),
        ],
        out_specs=pl.BlockSpec(memory_space=pltpu.SMEM),
        out_shape=jax.ShapeDtypeStruct((n_sample,), jnp.int32),
    )(xs, ys, zs, pos[:, 0], pos[:, 1], pos[:, 2])


QB = 64
DEPTH = 5
NCH = 128
CL = 128
R2 = RADIUS * RADIUS
BIGI = N_POINTS
INF = float("inf")


def _select_kernel(qx_ref, qy_ref, qz_ref, xs_ref, ys_ref, zs_ref,
                   nbr_ref, val_ref):
    qx = qx_ref[...].reshape(QB, 1, 1)
    qy = qy_ref[...].reshape(QB, 1, 1)
    qz = qz_ref[...].reshape(QB, 1, 1)
    xs = xs_ref[...].reshape(1, NCH, CL)
    ys = ys_ref[...].reshape(1, NCH, CL)
    zs = zs_ref[...].reshape(1, NCH, CL)

    lane3 = jax.lax.broadcasted_iota(jnp.int32, (QB, NCH, CL), 2)
    chunk2 = jax.lax.broadcasted_iota(jnp.int32, (QB, NCH), 1)
    lane32 = jax.lax.broadcasted_iota(jnp.int32, (QB, K), 1)

    def d2_full():
        dx = qx - xs
        dy = qy - ys
        dz = qz - zs
        return (dx * dx + dy * dy) + dz * dz

    D = d2_full()
    vals_l, idxs_l = [], []
    for _ in range(DEPTH):
        m = jnp.min(D, axis=2)
        eq = D == m[:, :, None]
        jl = jnp.min(jnp.where(eq, lane3, BIGI), axis=2)
        D = jnp.where(lane3 == jl[:, :, None], INF, D)
        vals_l.append(m)
        idxs_l.append(chunk2 * CL + jl)
    vals = jnp.stack(vals_l, axis=1)
    idxs = jnp.stack(idxs_l, axis=1)
    vlast = vals_l[-1]

    def p2_body(t, state):
        v, nbr, val, _ = state
        m2 = jnp.min(v, axis=(1, 2))
        eq = v == m2[:, None, None]
        j = jnp.min(jnp.where(eq, idxs, BIGI), axis=(1, 2))
        v = jnp.where(eq & (idxs == j[:, None, None]), INF, v)
        nbr = jnp.where(lane32 == t, j[:, None], nbr)
        ok = (m2 <= R2).astype(jnp.int32)
        val = jnp.where(lane32 == t, ok[:, None], val)
        return (v, nbr, val, m2)

    nbr0 = jnp.zeros((QB, K), jnp.int32)
    val0 = jnp.zeros((QB, K), jnp.int32)
    _, nbr, val, t32 = jax.lax.fori_loop(
        0, K, p2_body, (vals, nbr0, val0, jnp.zeros((QB,), jnp.float32)))
    nbr_ref[...] = nbr
    val_ref[...] = val

    unsafe = jnp.any(vlast <= t32[:, None])

    @pl.when(unsafe)
    def _fallback():
        lin = chunk2[:, :, None] * CL + lane3
        Df = d2_full()

        def fb_body(t, state):
            Dc, nbr, val = state
            m2 = jnp.min(Dc, axis=(1, 2))
            eq = Dc == m2[:, None, None]
            j = jnp.min(jnp.where(eq, lin, BIGI), axis=(1, 2))
            Dc = jnp.where(lin == j[:, None, None], INF, Dc)
            nbr = jnp.where(lane32 == t, j[:, None], nbr)
            ok = (m2 <= R2).astype(jnp.int32)
            val = jnp.where(lane32 == t, ok[:, None], val)
            return (Dc, nbr, val)

        _, nbr_f, val_f = jax.lax.fori_loop(0, K, fb_body, (Df, nbr0, val0))
        nbr_ref[...] = nbr_f
        val_ref[...] = val_f


def _radius_query(pos, q, r, k):
    s = q.shape[0]
    qx = q[:, 0].reshape(s, 1)
    qy = q[:, 1].reshape(s, 1)
    qz = q[:, 2].reshape(s, 1)
    xs = pos[:, 0].reshape(NCH, CL)
    ys = pos[:, 1].reshape(NCH, CL)
    zs = pos[:, 2].reshape(NCH, CL)
    grid = s // QB
    nbr, val = pl.pallas_call(
        _select_kernel,
        grid=(grid,),
        in_specs=[
            pl.BlockSpec((QB, 1), lambda i: (i, 0)),
            pl.BlockSpec((QB, 1), lambda i: (i, 0)),
            pl.BlockSpec((QB, 1), lambda i: (i, 0)),
            pl.BlockSpec((NCH, CL), lambda i: (0, 0)),
            pl.BlockSpec((NCH, CL), lambda i: (0, 0)),
            pl.BlockSpec((NCH, CL), lambda i: (0, 0)),
        ],
        out_specs=[
            pl.BlockSpec((QB, K), lambda i: (i, 0)),
            pl.BlockSpec((QB, K), lambda i: (i, 0)),
        ],
        out_shape=[
            jax.ShapeDtypeStruct((s, k), jnp.int32),
            jax.ShapeDtypeStruct((s, k), jnp.int32),
        ],
    )(qx, qy, qz, xs, ys, zs)
    return nbr, val != 0


PADW = 128
E = N_SAMPLE * K
SC_CHUNK = 512


def _sc_gather(pos16, col):
    info = plsc.get_sparse_core_info()
    nw = info.num_cores * info.num_subcores
    b_per_w = E // nw
    nch = b_per_w // SC_CHUNK
    mesh = plsc.VectorSubcoreMesh(core_axis_name="c", subcore_axis_name="s")

    @functools.partial(
        pl.kernel, mesh=mesh,
        out_type=jax.ShapeDtypeStruct((E, PADW), jnp.float32),
        scratch_types=[
            pltpu.VMEM((SC_CHUNK,), jnp.int32),
            pltpu.VMEM((SC_CHUNK, PADW), jnp.float32),
            pltpu.SemaphoreType.DMA,
        ],
    )
    def k(table_hbm, idx_hbm, out_hbm, idx_v, rows_v, sem):
        wid = lax.axis_index("s") * info.num_cores + lax.axis_index("c")
        base = wid * b_per_w
        for ch in range(nch):
            off = base + ch * SC_CHUNK
            pltpu.sync_copy(idx_hbm.at[pl.ds(off, SC_CHUNK)], idx_v)
            pltpu.async_copy(table_hbm.at[idx_v], rows_v, sem).wait()
            pltpu.sync_copy(rows_v, out_hbm.at[pl.ds(off, SC_CHUNK)])

    return k(pos16, col)


def _edge_kernel(relj_ref, reli_ref, mask_ref, w1_ref, b1_ref, w2_ref, b2_ref,
                 out_ref):
    rel = relj_ref[:, :3] - reli_ref[...]
    h = jnp.maximum(rel @ w1_ref[...] + b1_ref[...], 0.0)
    msg = h @ w2_ref[...] + b2_ref[...]
    neg = jnp.float32(-jnp.inf)
    msg = jnp.where(mask_ref[...], msg, neg)
    msg = msg.reshape(QBLK, K, OUT)
    red = jnp.max(msg, axis=1)
    out_ref[...] = jnp.where(jnp.isfinite(red), red, 0.0)


def kernel(pos, batch, W1, b1, W2, b2):
    idx = _fps(pos, N_SAMPLE)
    q = pos[idx]
    nbr, valid = _radius_query(pos, q, RADIUS, K)

    col = nbr.reshape(-1)
    pos16 = jnp.pad(pos, ((0, 0), (0, PADW - 3)))
    posj = _sc_gather(pos16, col)
    posi = jnp.repeat(pos[:N_SAMPLE], K, axis=0)
    mask = valid.reshape(-1, 1)

    grid = N_SAMPLE // QBLK
    out_top = pl.pallas_call(
        _edge_kernel,
        grid=(grid,),
        in_specs=[
            pl.BlockSpec((QBLK * K, PADW), lambda i: (i, 0)),
            pl.BlockSpec((QBLK * K, 3), lambda i: (i, 0)),
            pl.BlockSpec((QBLK * K, 1), lambda i: (i, 0)),
            pl.BlockSpec((3, HIDDEN), lambda i: (0, 0)),
            pl.BlockSpec((HIDDEN,), lambda i: (0,)),
            pl.BlockSpec((HIDDEN, OUT), lambda i: (0, 0)),
            pl.BlockSpec((OUT,), lambda i: (0,)),
        ],
        out_specs=pl.BlockSpec((QBLK, OUT), lambda i: (i, 0)),
        out_shape=jax.ShapeDtypeStruct((N_SAMPLE, OUT), jnp.float32),
    )(posj, posi, mask, W1, b1, W2, b2)

    return jnp.concatenate(
        [out_top, jnp.zeros((N_POINTS - N_SAMPLE, OUT), jnp.float32)], axis=0)


# speedup vs baseline: 1.1821x; 1.0073x over previous
import functools

import jax
import jax.numpy as jnp
from jax import lax
from jax.experimental import pallas as pl
from jax.experimental.pallas import tpu as pltpu
from jax.experimental.pallas import tpu_sc as plsc

N_POINTS = 16384
N_SAMPLE = 8192
RADIUS = 0.2
K = 32
HIDDEN = 64
OUT = 128

QBLK = 256


FPS_R = 128
FPS_C = 128


def _fps_kernel(xs_ref, ys_ref, zs_ref, xsm_ref, ysm_ref, zsm_ref, sel_ref):
    rows = jax.lax.broadcasted_iota(jnp.int32, (FPS_R, FPS_C), 0)
    cols = jax.lax.broadcasted_iota(jnp.int32, (FPS_R, FPS_C), 1)
    lin = rows * FPS_C + cols
    xs = xs_ref[...]
    ys = ys_ref[...]
    zs = zs_ref[...]
    sel_ref[0] = 0

    def body(i, state):
        last, dists = state
        px = xsm_ref[last]
        py = ysm_ref[last]
        pz = zsm_ref[last]
        dx = xs - px
        dy = ys - py
        dz = zs - pz
        d = (dx * dx + dz * dz) + dy * dy
        dists = jnp.minimum(dists, d)
        nxt = jnp.argmax(dists).astype(jnp.int32)
        sel_ref[i] = nxt
        return (nxt, dists)

    dists0 = jnp.full((FPS_R, FPS_C), jnp.inf, dtype=jnp.float32)
    jax.lax.fori_loop(1, N_SAMPLE, body, (jnp.int32(0), dists0))


def _fps(pos, n_sample):
    xs = pos[:, 0].reshape(FPS_R, FPS_C)
    ys = pos[:, 1].reshape(FPS_R, FPS_C)
    zs = pos[:, 2].reshape(FPS_R, FPS_C)
    return pl.pallas_call(
        _fps_kernel,
        in_specs=[
            pl.BlockSpec(memory_space=pltpu.VMEM),
            pl.BlockSpec(memory_space=pltpu.VMEM),
            pl.BlockSpec(memory_space=pltpu.VMEM),
            pl.BlockSpec(memory_space=pltpu.SMEM),
            pl.BlockSpec(memory_space=pltpu.SMEM),
            pl.BlockSpec(memory_space=pltpu.SMEM),
        ],
        out_specs=pl.BlockSpec(memory_space=pltpu.SMEM),
        out_shape=jax.ShapeDtypeStruct((n_sample,), jnp.int32),
    )(xs, ys, zs, pos[:, 0], pos[:, 1], pos[:, 2])


QB = 64
DEPTH = 5
NCH = 128
CL = 128
R2 = RADIUS * RADIUS
BIGI = N_POINTS
INF = float("inf")


def _select_kernel(qx_ref, qy_ref, qz_ref, xs_ref, ys_ref, zs_ref,
                   nbr_ref, val_ref):
    qx = qx_ref[...].reshape(QB, 1, 1)
    qy = qy_ref[...].reshape(QB, 1, 1)
    qz = qz_ref[...].reshape(QB, 1, 1)
    xs = xs_ref[...].reshape(1, NCH, CL)
    ys = ys_ref[...].reshape(1, NCH, CL)
    zs = zs_ref[...].reshape(1, NCH, CL)

    lane3 = jax.lax.broadcasted_iota(jnp.int32, (QB, NCH, CL), 2)
    chunk2 = jax.lax.broadcasted_iota(jnp.int32, (QB, NCH), 1)
    lane32 = jax.lax.broadcasted_iota(jnp.int32, (QB, K), 1)

    def d2_full():
        dx = qx - xs
        dy = qy - ys
        dz = qz - zs
        return (dx * dx + dy * dy) + dz * dz

    D = d2_full()
    vals_l, idxs_l = [], []
    for _ in range(DEPTH):
        m = jnp.min(D, axis=2)
        eq = D == m[:, :, None]
        jl = jnp.min(jnp.where(eq, lane3, BIGI), axis=2)
        D = jnp.where(lane3 == jl[:, :, None], INF, D)
        vals_l.append(m)
        idxs_l.append(chunk2 * CL + jl)
    vals = jnp.stack(vals_l, axis=1)
    idxs = jnp.stack(idxs_l, axis=1)
    vlast = vals_l[-1]

    def p2_body(t, state):
        v, nbr, val, _ = state
        m2 = jnp.min(v, axis=(1, 2))
        eq = v == m2[:, None, None]
        j = jnp.min(jnp.where(eq, idxs, BIGI), axis=(1, 2))
        v = jnp.where(eq & (idxs == j[:, None, None]), INF, v)
        nbr = jnp.where(lane32 == t, j[:, None], nbr)
        ok = (m2 <= R2).astype(jnp.int32)
        val = jnp.where(lane32 == t, ok[:, None], val)
        return (v, nbr, val, m2)

    nbr0 = jnp.zeros((QB, K), jnp.int32)
    val0 = jnp.zeros((QB, K), jnp.int32)
    _, nbr, val, t32 = jax.lax.fori_loop(
        0, K, p2_body, (vals, nbr0, val0, jnp.zeros((QB,), jnp.float32)))
    nbr_ref[...] = nbr
    val_ref[...] = val

    unsafe = jnp.any(vlast <= t32[:, None])

    @pl.when(unsafe)
    def _fallback():
        lin = chunk2[:, :, None] * CL + lane3
        Df = d2_full()

        def fb_body(t, state):
            Dc, nbr, val = state
            m2 = jnp.min(Dc, axis=(1, 2))
            eq = Dc == m2[:, None, None]
            j = jnp.min(jnp.where(eq, lin, BIGI), axis=(1, 2))
            Dc = jnp.where(lin == j[:, None, None], INF, Dc)
            nbr = jnp.where(lane32 == t, j[:, None], nbr)
            ok = (m2 <= R2).astype(jnp.int32)
            val = jnp.where(lane32 == t, ok[:, None], val)
            return (Dc, nbr, val)

        _, nbr_f, val_f = jax.lax.fori_loop(0, K, fb_body, (Df, nbr0, val0))
        nbr_ref[...] = nbr_f
        val_ref[...] = val_f


def _radius_query(pos, q, r, k):
    s = q.shape[0]
    qx = q[:, 0].reshape(s, 1)
    qy = q[:, 1].reshape(s, 1)
    qz = q[:, 2].reshape(s, 1)
    xs = pos[:, 0].reshape(NCH, CL)
    ys = pos[:, 1].reshape(NCH, CL)
    zs = pos[:, 2].reshape(NCH, CL)
    grid = s // QB
    nbr, val = pl.pallas_call(
        _select_kernel,
        grid=(grid,),
        in_specs=[
            pl.BlockSpec((QB, 1), lambda i: (i, 0)),
            pl.BlockSpec((QB, 1), lambda i: (i, 0)),
            pl.BlockSpec((QB, 1), lambda i: (i, 0)),
            pl.BlockSpec((NCH, CL), lambda i: (0, 0)),
            pl.BlockSpec((NCH, CL), lambda i: (0, 0)),
            pl.BlockSpec((NCH, CL), lambda i: (0, 0)),
        ],
        out_specs=[
            pl.BlockSpec((QB, K), lambda i: (i, 0)),
            pl.BlockSpec((QB, K), lambda i: (i, 0)),
        ],
        out_shape=[
            jax.ShapeDtypeStruct((s, k), jnp.int32),
            jax.ShapeDtypeStruct((s, k), jnp.int32),
        ],
    )(qx, qy, qz, xs, ys, zs)
    return nbr, val != 0


PADW = 128
E = N_SAMPLE * K
SC_CHUNK = 512


def _sc_gather(pos16, col):
    info = plsc.get_sparse_core_info()
    nw = info.num_cores * info.num_subcores
    b_per_w = E // nw
    nch = b_per_w // SC_CHUNK
    mesh = plsc.VectorSubcoreMesh(core_axis_name="c", subcore_axis_name="s")

    @functools.partial(
        pl.kernel, mesh=mesh,
        out_type=jax.ShapeDtypeStruct((E, PADW), jnp.float32),
        scratch_types=[
            pltpu.VMEM((SC_CHUNK,), jnp.int32),
            pltpu.VMEM((SC_CHUNK, PADW), jnp.float32),
            pltpu.SemaphoreType.DMA,
        ],
    )
    def k(table_hbm, idx_hbm, out_hbm, idx_v, rows_v, sem):
        wid = lax.axis_index("s") * info.num_cores + lax.axis_index("c")
        base = wid * b_per_w
        for ch in range(nch):
            off = base + ch * SC_CHUNK
            pltpu.sync_copy(idx_hbm.at[pl.ds(off, SC_CHUNK)], idx_v)
            pltpu.async_copy(table_hbm.at[idx_v], rows_v, sem).wait()
            pltpu.sync_copy(rows_v, out_hbm.at[pl.ds(off, SC_CHUNK)])

    return k(pos16, col)


def _edge_kernel(relj_ref, reli_ref, mask_ref, w1_ref, b1_ref, w2_ref, b2_ref,
                 out_ref):
    rel = relj_ref[:, :3] - reli_ref[...]
    h = jnp.maximum(rel @ w1_ref[...] + b1_ref[...], 0.0)
    msg = h @ w2_ref[...] + b2_ref[...]
    neg = jnp.float32(-jnp.inf)
    msg = jnp.where(mask_ref[...], msg, neg)
    msg = msg.reshape(QBLK, K, OUT)
    red = jnp.max(msg, axis=1)
    out_ref[...] = jnp.where(jnp.isfinite(red), red, 0.0)


def kernel(pos, batch, W1, b1, W2, b2):
    idx = _fps(pos, N_SAMPLE)
    q = pos[idx]
    nbr, valid = _radius_query(pos, q, RADIUS, K)

    col = nbr.reshape(-1)
    pos16 = jnp.pad(pos, ((0, 0), (0, PADW - 3)))
    posj = _sc_gather(pos16, col)
    posi = jnp.repeat(pos[:N_SAMPLE], K, axis=0)
    mask = valid.reshape(-1, 1)

    grid = N_SAMPLE // QBLK
    out_top = pl.pallas_call(
        _edge_kernel,
        grid=(grid,),
        in_specs=[
            pl.BlockSpec((QBLK * K, PADW), lambda i: (i, 0)),
            pl.BlockSpec((QBLK * K, 3), lambda i: (i, 0)),
            pl.BlockSpec((QBLK * K, 1), lambda i: (i, 0)),
            pl.BlockSpec((3, HIDDEN), lambda i: (0, 0)),
            pl.BlockSpec((HIDDEN,), lambda i: (0,)),
            pl.BlockSpec((HIDDEN, OUT), lambda i: (0, 0)),
            pl.BlockSpec((OUT,), lambda i: (0,)),
        ],
        out_specs=pl.BlockSpec((QBLK, OUT), lambda i: (i, 0)),
        out_shape=jax.ShapeDtypeStruct((N_SAMPLE, OUT), jnp.float32),
    )(posj, posi, mask, W1, b1, W2, b2)

    return jnp.concatenate(
        [out_top, jnp.zeros((N_POINTS - N_SAMPLE, OUT), jnp.float32)], axis=0)
